# Initial kernel scaffold; baseline (speedup 1.0000x reference)
import jax
import jax.numpy as jnp
from jax.experimental import pallas as pl


def kernel(z, W):
    raise NotImplementedError("write your pallas kernel here")


# speedup vs baseline: 1.0832x; 1.0832x over previous
import functools

import jax
import jax.numpy as jnp
from jax import lax
from jax.experimental import pallas as pl
from jax.experimental.pallas import tpu as pltpu
from jax.experimental.pallas import tpu_sc as plsc

N_E = 8192
D = 64
M = 36864
BM = 512
BN = 512
NM = M // BM
NN = N_E // BN
BETA = 0.25

NC = 2
NS = 16
NW = NC * NS
RPW = M // NW
G = 128
NCH = RPW // G


def _argmin_body(z_ref, w_ref, idx_ref, loss_ref, acc_ref):
    i = pl.program_id(0)
    zb = z_ref[...]
    t1 = jnp.sum(zb * zb, axis=1, keepdims=True)
    run_min = jnp.full((BM, 1), jnp.inf, jnp.float32)
    run_loss = jnp.full((BM, 1), jnp.inf, jnp.float32)
    run_idx = jnp.zeros((BM, 1), jnp.int32)
    for n in range(NN):
        wb = w_ref[pl.ds(n * BN, BN), :]
        t2 = jnp.sum(wb * wb, axis=1)
        t3 = lax.dot_general(zb, wb, (((1,), (1,)), ((), ())),
                             preferred_element_type=jnp.float32)
        d = (t1 + t2[None, :]) - 2.0 * t3
        bmin = jnp.min(d, axis=1, keepdims=True)
        iota = lax.broadcasted_iota(jnp.int32, (BM, BN), 1) + (n * BN)
        bidx = jnp.min(jnp.where(d == bmin, iota, jnp.int32(2**30)),
                       axis=1, keepdims=True)
        better = bmin < run_min
        run_idx = jnp.where(better, bidx, run_idx)
        run_min = jnp.where(better, bmin, run_min)
        run_loss = jnp.minimum(run_loss, bmin)
        if (n + 1) * BN == N_E // 2:
            run_min = run_min.astype(jnp.bfloat16).astype(jnp.float32)
    idx_ref[0, 0, :] = run_idx[:, 0]

    @pl.when(i == 0)
    def _():
        acc_ref[0] = 0.0

    acc_ref[0] += jnp.sum(run_loss)

    @pl.when(i == NM - 1)
    def _():
        loss_ref[0, 0] = acc_ref[0] * ((1.0 + BETA) / (M * D))


def _argmin_call(z_flat, W):
    return pl.pallas_call(
        _argmin_body,
        grid=(NM,),
        in_specs=[
            pl.BlockSpec((BM, D), lambda i: (i, 0)),
            pl.BlockSpec((N_E, D), lambda i: (0, 0)),
        ],
        out_specs=[
            pl.BlockSpec((1, 1, BM), lambda i: (i, 0, 0)),
            pl.BlockSpec(memory_space=pltpu.SMEM),
        ],
        out_shape=[
            jax.ShapeDtypeStruct((NM, 1, BM), jnp.int32),
            jax.ShapeDtypeStruct((1, 1), jnp.float32),
        ],
        scratch_shapes=[pltpu.SMEM((1,), jnp.float32)],
    )(z_flat, W)


def _sc_gather_hist_body(idx_hbm, w_hbm, zq_hbm, hist_hbm,
                         idx_v, rows_v, hist_v, sem):
    c = lax.axis_index("c")
    s = lax.axis_index("s")
    wid = s * NC + c
    base = wid * RPW
    pltpu.sync_copy(idx_hbm.at[pl.ds(base, RPW)], idx_v)

    zero16 = jnp.zeros((16,), jnp.float32)

    def _zero(i, carry):
        hist_v[pl.ds(i * 16, 16)] = zero16
        return carry

    lax.fori_loop(0, N_E // 16, _zero, 0)

    ones16 = jnp.ones((16,), jnp.float32)
    for j in range(NCH):
        pltpu.async_copy(w_hbm.at[idx_v.at[pl.ds(j * G, G)]], rows_v, sem).wait()
        pltpu.sync_copy(rows_v, zq_hbm.at[pl.ds(base + j * G, G)])
        for t in range(G // 16):
            v = idx_v[pl.ds(j * G + t * 16, 16)]
            plsc.addupdate_scatter(hist_v, [v], ones16)
    pltpu.sync_copy(hist_v, hist_hbm.at[wid])


def _sc_gather_hist(idx_flat, W):
    mesh = plsc.VectorSubcoreMesh(core_axis_name="c", subcore_axis_name="s")
    f = functools.partial(
        pl.kernel,
        mesh=mesh,
        out_type=[
            jax.ShapeDtypeStruct((M, D), jnp.float32),
            jax.ShapeDtypeStruct((NW, N_E), jnp.float32),
        ],
        scratch_types=[
            pltpu.VMEM((RPW,), jnp.int32),
            pltpu.VMEM((G, D), jnp.float32),
            pltpu.VMEM((N_E,), jnp.float32),
            pltpu.SemaphoreType.DMA,
        ],
        compiler_params=pltpu.CompilerParams(
            needs_layout_passes=False, use_tc_tiling_on_sc=False),
    )(_sc_gather_hist_body)
    return f(idx_flat, W)


def _finalize_body(hist_ref, out_ref):
    counts = jnp.sum(hist_ref[...], axis=0)
    p = counts * jnp.float32(1.0 / M)
    ent = p * jnp.log(p + jnp.float32(1e-10))
    out_ref[0, 0] = jnp.exp(-jnp.sum(ent))


def _finalize_call(hists):
    return pl.pallas_call(
        _finalize_body,
        in_specs=[pl.BlockSpec((NW, N_E), lambda: (0, 0))],
        out_specs=pl.BlockSpec(memory_space=pltpu.SMEM),
        out_shape=jax.ShapeDtypeStruct((1, 1), jnp.float32),
    )(hists)


def kernel(z, W):
    B, k, d_ = z.shape
    z_flat = z.reshape(B * k, d_)
    idx3, loss11 = _argmin_call(z_flat, W)
    idx_flat = idx3.reshape(M)
    zq, hists = _sc_gather_hist(idx_flat, W)
    perp11 = _finalize_call(hists)
    return (loss11[0, 0], zq.reshape(B, k, d_), perp11[0, 0],
            idx3.reshape(B, k))


# speedup vs baseline: 2.1954x; 2.0267x over previous
import functools

import jax
import jax.numpy as jnp
from jax import lax
from jax.experimental import pallas as pl
from jax.experimental.pallas import tpu as pltpu
from jax.experimental.pallas import tpu_sc as plsc

N_E = 8192
D = 64
M = 36864
BM = 512
BN = 512
NM = M // BM
NN = N_E // BN
BETA = 0.25

NC = 2
NS = 16
NW = NC * NS
RPW = M // NW
G = 128
NCH = RPW // G


def _extract(acc_v, acc_i):
    bmin = jnp.min(acc_v, axis=1, keepdims=True)
    bidx = jnp.min(jnp.where(acc_v == bmin, acc_i, jnp.float32(1e9)),
                   axis=1, keepdims=True)
    return bmin, bidx


def _argmin_body(z_ref, w_ref, idx_ref, loss_ref, t2_ref, acc_ref):
    i = pl.program_id(0)

    @pl.when(i == 0)
    def _():
        wall = w_ref[...]
        t2_ref[0, :] = jnp.sum(wall * wall, axis=1)
        acc_ref[0] = 0.0

    zb = z_ref[...]
    t1 = jnp.sum(zb * zb, axis=1, keepdims=True)
    z2 = zb + zb
    lane = lax.broadcasted_iota(jnp.int32, (1, 128), 1).astype(jnp.float32)
    acc_v = jnp.full((BM, 128), jnp.inf, jnp.float32)
    acc_i = jnp.zeros((BM, 128), jnp.float32)
    for n in range(NN):
        wb = w_ref[pl.ds(n * BN, BN), :]
        t3b = lax.dot_general(z2, wb, (((1,), (1,)), ((), ())),
                              preferred_element_type=jnp.float32)
        for v in range(BN // 128):
            t2v = t2_ref[0, pl.ds(n * BN + v * 128, 128)][None, :]
            dv = (t1 + t2v) - t3b[:, v * 128:(v + 1) * 128]
            cm = dv < acc_v
            acc_v = jnp.where(cm, dv, acc_v)
            acc_i = jnp.where(cm, lane + jnp.float32(n * BN + v * 128), acc_i)
        if (n + 1) * BN == N_E // 2:
            m1, i1 = _extract(acc_v, acc_i)
            m1r = m1.astype(jnp.bfloat16).astype(jnp.float32)
            acc_v = jnp.broadcast_to(m1r, (BM, 128))
            acc_i = jnp.broadcast_to(i1, (BM, 128))
    fmin, fidx = _extract(acc_v, acc_i)
    idx_ref[0, 0, :] = fidx[:, 0].astype(jnp.int32)
    acc_ref[0] += jnp.sum(fmin)

    @pl.when(i == NM - 1)
    def _():
        loss_ref[0, 0] = acc_ref[0] * ((1.0 + BETA) / (M * D))


def _argmin_call(z_flat, W):
    return pl.pallas_call(
        _argmin_body,
        grid=(NM,),
        in_specs=[
            pl.BlockSpec((BM, D), lambda i: (i, 0)),
            pl.BlockSpec((N_E, D), lambda i: (0, 0)),
        ],
        out_specs=[
            pl.BlockSpec((1, 1, BM), lambda i: (i, 0, 0)),
            pl.BlockSpec(memory_space=pltpu.SMEM),
        ],
        out_shape=[
            jax.ShapeDtypeStruct((NM, 1, BM), jnp.int32),
            jax.ShapeDtypeStruct((1, 1), jnp.float32),
        ],
        scratch_shapes=[pltpu.VMEM((1, N_E), jnp.float32),
                        pltpu.SMEM((1,), jnp.float32)],
    )(z_flat, W)


def _sc_gather_hist_body(idx_hbm, w_hbm, zq_hbm, hist_hbm,
                         idx_v, rows_v, hist_v, sem):
    c = lax.axis_index("c")
    s = lax.axis_index("s")
    wid = s * NC + c
    base = wid * RPW
    pltpu.sync_copy(idx_hbm.at[pl.ds(base, RPW)], idx_v)

    zero16 = jnp.zeros((16,), jnp.float32)

    def _zero(i, carry):
        hist_v[pl.ds(i * 16, 16)] = zero16
        return carry

    lax.fori_loop(0, N_E // 16, _zero, 0)

    ones16 = jnp.ones((16,), jnp.float32)
    for j in range(NCH):
        pltpu.async_copy(w_hbm.at[idx_v.at[pl.ds(j * G, G)]], rows_v, sem).wait()
        pltpu.sync_copy(rows_v, zq_hbm.at[pl.ds(base + j * G, G)])
        for t in range(G // 16):
            v = idx_v[pl.ds(j * G + t * 16, 16)]
            plsc.addupdate_scatter(hist_v, [v], ones16)
    pltpu.sync_copy(hist_v, hist_hbm.at[wid])


def _sc_gather_hist(idx_flat, W):
    mesh = plsc.VectorSubcoreMesh(core_axis_name="c", subcore_axis_name="s")
    f = functools.partial(
        pl.kernel,
        mesh=mesh,
        out_type=[
            jax.ShapeDtypeStruct((M, D), jnp.float32),
            jax.ShapeDtypeStruct((NW, N_E), jnp.float32),
        ],
        scratch_types=[
            pltpu.VMEM((RPW,), jnp.int32),
            pltpu.VMEM((G, D), jnp.float32),
            pltpu.VMEM((N_E,), jnp.float32),
            pltpu.SemaphoreType.DMA,
        ],
        compiler_params=pltpu.CompilerParams(
            needs_layout_passes=False, use_tc_tiling_on_sc=False),
    )(_sc_gather_hist_body)
    return f(idx_flat, W)


def _finalize_body(hist_ref, out_ref):
    counts = jnp.sum(hist_ref[...], axis=0)
    p = counts * jnp.float32(1.0 / M)
    ent = p * jnp.log(p + jnp.float32(1e-10))
    out_ref[0, 0] = jnp.exp(-jnp.sum(ent))


def _finalize_call(hists):
    return pl.pallas_call(
        _finalize_body,
        in_specs=[pl.BlockSpec((NW, N_E), lambda: (0, 0))],
        out_specs=pl.BlockSpec(memory_space=pltpu.SMEM),
        out_shape=jax.ShapeDtypeStruct((1, 1), jnp.float32),
    )(hists)


def kernel(z, W):
    B, k, d_ = z.shape
    z_flat = z.reshape(B * k, d_)
    idx3, loss11 = _argmin_call(z_flat, W)
    idx_flat = idx3.reshape(M)
    zq, hists = _sc_gather_hist(idx_flat, W)
    perp11 = _finalize_call(hists)
    return (loss11[0, 0], zq.reshape(B, k, d_), perp11[0, 0],
            idx3.reshape(B, k))
